# Initial kernel scaffold; baseline (speedup 1.0000x reference)
import jax
import jax.numpy as jnp
from jax.experimental import pallas as pl


def kernel(x, edge_index, W1, b1, W2, b2, W3, b3, W4, b4, W5, b5, g2, be2, g3, be3):
    raise NotImplementedError("write your pallas kernel here")


# speedup vs baseline: 7.6167x; 7.6167x over previous
import functools

import jax
import jax.numpy as jnp
from jax import lax
from jax.experimental import pallas as pl
from jax.experimental.pallas import tpu as pltpu
from jax.experimental.pallas import tpu_sc as plsc

_SC_PARAMS = pltpu.CompilerParams(use_tc_tiling_on_sc=False)

_NC, _NS = 2, 16
_CHUNK = 128
_K = 8


def _make_mesh():
    return plsc.VectorSubcoreMesh(
        core_axis_name="c", subcore_axis_name="s",
        num_cores=_NC, num_subcores=_NS)


def _make_agg(half, f, n_chunks):
    n_acc = half + 16
    cpt = n_chunks // _NS
    groups = cpt // _K
    rpt = n_acc // _NS
    opt = half // _NS

    @functools.partial(
        pl.kernel,
        out_type=jax.ShapeDtypeStruct((_NC, half, f), jnp.float32),
        mesh=_make_mesh(),
        compiler_params=_SC_PARAMS,
        scratch_types=[
            pltpu.VMEM((_K, _CHUNK), jnp.int32),
            pltpu.VMEM((_K, _CHUNK), jnp.int32),
            pltpu.VMEM((_K, _CHUNK, f), jnp.float32),
            pltpu.VMEM((rpt, f), jnp.float32),
            pltpu.VMEM_SHARED((n_acc, f), jnp.float32),
            pltpu.SemaphoreType.DMA,
        ],
    )
    def agg(y_hbm, src_hbm, dstl_hbm, zero_hbm, out_hbm,
            sidx, didx, rows, tbuf, acc, sem):
        cid = lax.axis_index("c")
        sid = lax.axis_index("s")
        pltpu.sync_copy(zero_hbm, tbuf)
        pltpu.sync_copy(tbuf, acc.at[pl.ds(sid * rpt, rpt)])
        plsc.subcore_barrier()

        def body(g, carry):
            c0 = sid * cpt + g * _K
            pltpu.sync_copy(src_hbm.at[pl.ds(c0, _K)], sidx)
            pltpu.sync_copy(dstl_hbm.at[cid, pl.ds(c0, _K)], didx)
            cps = [pltpu.async_copy(y_hbm.at[sidx.at[j]], rows.at[j], sem)
                   for j in range(_K)]
            for cp in cps:
                cp.wait()
            for j in range(_K):
                pltpu.sync_copy(rows.at[j], acc.at[didx.at[j]], add=True)
            return carry

        lax.fori_loop(0, groups, body, 0)
        plsc.subcore_barrier()
        r0 = sid * opt
        pltpu.sync_copy(acc.at[pl.ds(r0, opt)], tbuf.at[pl.ds(0, opt)])
        pltpu.sync_copy(tbuf.at[pl.ds(0, opt)], out_hbm.at[cid, pl.ds(r0, opt)])

    return agg


def _elu(t):
    return jnp.where(t > 0, t, jnp.exp(jnp.minimum(t, 0.0)) - 1.0)


def _tc_dstloc(half, dst_ref, dstl_ref):
    d = dst_ref[...]
    dstl_ref[0] = jnp.where(d < half, d, half)
    dstl_ref[1] = jnp.where(d >= half, d - half, half)


def _tc_prep(dp_ref, x_ref, w1_ref, dinv_ref, y1_ref):
    deg = dp_ref[:, 0:1] + 1.0
    dinv = lax.rsqrt(deg)
    dinv_ref[...] = dinv
    xw = jnp.dot(x_ref[...], w1_ref[...], preferred_element_type=jnp.float32)
    y1_ref[...] = dinv * xw


def _tc_mid(a_ref, y_ref, dinv_ref, b_ref, g_ref, be_ref, wn_ref, yn_ref):
    dinv = dinv_ref[...]
    t = dinv * (a_ref[...] + y_ref[...]) + b_ref[...]
    m = jnp.mean(t, axis=-1, keepdims=True)
    d = t - m
    v = jnp.mean(d * d, axis=-1, keepdims=True)
    h = _elu(d * lax.rsqrt(v + 1e-5) * g_ref[...] + be_ref[...])
    yn_ref[...] = dinv * jnp.dot(h, wn_ref[...],
                                 preferred_element_type=jnp.float32)


def _tc_l4(a_ref, y_ref, dinv_ref, b4_ref, w5_ref, y5_ref):
    dinv = dinv_ref[...]
    t = dinv * (a_ref[...] + y_ref[...]) + b4_ref[...]
    h = _elu(t)
    y5 = dinv * jnp.sum(h * w5_ref[...], axis=-1, keepdims=True)
    y5_ref[...] = y5 + jnp.zeros((1, 8), jnp.float32)


def _tc_l5(a_ref, y_ref, dinv_ref, b5_ref, out_ref):
    t = dinv_ref[...] * (a_ref[:, 0:1] + y_ref[:, 0:1]) + b5_ref[...]
    out_ref[...] = _elu(t)


def _tc_call(body, out_shape, *args):
    return pl.pallas_call(body, out_shape=out_shape)(*args)


def kernel(x, edge_index, W1, b1, W2, b2, W3, b3, W4, b4, W5, b5,
           g2, be2, g3, be3):
    n, fin = x.shape
    e = edge_index.shape[1]
    f1 = W1.shape[1]
    f4 = W4.shape[1]
    n_pad = (n // 256 + 1) * 256
    half = n_pad // 2
    n_chunks = -(-e // _CHUNK)
    n_chunks = -(-n_chunks // (_NS * _K)) * (_NS * _K)
    e_pad = n_chunks * _CHUNK

    src = jnp.pad(edge_index[0], (0, e_pad - e), constant_values=n)
    dst = jnp.pad(edge_index[1], (0, e_pad - e), constant_values=n)
    src = src.reshape(n_chunks, _CHUNK)
    dst = dst.reshape(n_chunks, _CHUNK)

    x_pad = jnp.pad(x, ((0, n_pad - n), (0, 128 - fin)))
    w1_pad = jnp.pad(W1, ((0, 128 - fin), (0, 0)))

    zf = jnp.zeros((half // _NS + 1, f1), jnp.float32)
    z4 = jnp.zeros((half // _NS + 1, f4), jnp.float32)
    z8 = jnp.zeros((half // _NS + 1, 8), jnp.float32)

    agg_f = _make_agg(half, f1, n_chunks)
    agg_4 = _make_agg(half, f4, n_chunks)
    agg_8 = _make_agg(half, 8, n_chunks)

    dstl = _tc_call(functools.partial(_tc_dstloc, half),
                    jax.ShapeDtypeStruct((_NC, n_chunks, _CHUNK), jnp.int32),
                    dst)

    ones_tab = jnp.ones((n_pad, 8), jnp.float32)
    dp = agg_8(ones_tab, src, dstl, z8).reshape(n_pad, 8)
    dinv, y1 = _tc_call(
        _tc_prep,
        (jax.ShapeDtypeStruct((n_pad, 1), jnp.float32),
         jax.ShapeDtypeStruct((n_pad, f1), jnp.float32)),
        dp, x_pad, w1_pad)

    g2r, be2r = g2[None, :], be2[None, :]
    g3r, be3r = g3[None, :], be3[None, :]

    a1 = agg_f(y1, src, dstl, zf).reshape(n_pad, f1)
    y2 = _tc_call(_tc_mid, jax.ShapeDtypeStruct((n_pad, f1), jnp.float32),
                  a1, y1, dinv, b1[None, :], g2r, be2r, W2)
    a2 = agg_f(y2, src, dstl, zf).reshape(n_pad, f1)
    y3 = _tc_call(_tc_mid, jax.ShapeDtypeStruct((n_pad, f1), jnp.float32),
                  a2, y2, dinv, b2[None, :], g2r, be2r, W3)
    a3 = agg_f(y3, src, dstl, zf).reshape(n_pad, f1)
    y4 = _tc_call(_tc_mid, jax.ShapeDtypeStruct((n_pad, f4), jnp.float32),
                  a3, y3, dinv, b3[None, :], g3r, be3r, W4)
    a4 = agg_4(y4, src, dstl, z4).reshape(n_pad, f4)
    y5 = _tc_call(_tc_l4, jax.ShapeDtypeStruct((n_pad, 8), jnp.float32),
                  a4, y4, dinv, b4[None, :], W5.reshape(1, f4))
    a5 = agg_8(y5, src, dstl, z8).reshape(n_pad, 8)
    h5 = _tc_call(_tc_l5, jax.ShapeDtypeStruct((n_pad, 1), jnp.float32),
                  a5, y5, dinv, b5[None, :])

    r = 100 if n == 10000 else int(n ** 0.5)
    return h5[:n, 0].reshape(r, r)[None, :, :]


# speedup vs baseline: 7.7964x; 1.0236x over previous
import functools

import jax
import jax.numpy as jnp
from jax import lax
from jax.experimental import pallas as pl
from jax.experimental.pallas import tpu as pltpu
from jax.experimental.pallas import tpu_sc as plsc

_SC_PARAMS = pltpu.CompilerParams(use_tc_tiling_on_sc=False)

_NC, _NS = 2, 16
_CHUNK = 128
_K = 4


def _make_mesh():
    return plsc.VectorSubcoreMesh(
        core_axis_name="c", subcore_axis_name="s",
        num_cores=_NC, num_subcores=_NS)


def _make_agg(half, f, n_chunks):
    n_acc = half + 16
    cpt = n_chunks // _NS
    groups = cpt // _K
    gpairs = groups // 2
    rpt = n_acc // _NS
    opt = half // _NS

    @functools.partial(
        pl.kernel,
        out_type=jax.ShapeDtypeStruct((_NC, half, f), jnp.float32),
        mesh=_make_mesh(),
        compiler_params=_SC_PARAMS,
        scratch_types=[
            pltpu.VMEM((_K, _CHUNK), jnp.int32),
            pltpu.VMEM((_K, _CHUNK), jnp.int32),
            pltpu.VMEM((_K, _CHUNK), jnp.int32),
            pltpu.VMEM((_K, _CHUNK), jnp.int32),
            pltpu.VMEM((_K, _CHUNK, f), jnp.float32),
            pltpu.VMEM((_K, _CHUNK, f), jnp.float32),
            pltpu.VMEM((rpt, f), jnp.float32),
            pltpu.VMEM_SHARED((n_acc, f), jnp.float32),
            pltpu.SemaphoreType.DMA,
            pltpu.SemaphoreType.DMA,
            pltpu.SemaphoreType.DMA,
        ],
    )
    def agg(y_hbm, src_hbm, dstl_hbm, zero_hbm, out_hbm,
            sidx0, sidx1, didx0, didx1, rows0, rows1, tbuf, acc,
            sem_g, sem_s0, sem_s1):
        cid = lax.axis_index("c")
        sid = lax.axis_index("s")
        pltpu.sync_copy(zero_hbm, tbuf)
        pltpu.sync_copy(tbuf, acc.at[pl.ds(sid * rpt, rpt)])
        plsc.subcore_barrier()
        sidxs, didxs, rowss = (sidx0, sidx1), (didx0, didx1), (rows0, rows1)
        sems = (sem_s0, sem_s1)

        def gather_group(g, b):
            c0 = sid * cpt + g * _K
            pltpu.sync_copy(src_hbm.at[pl.ds(c0, _K)], sidxs[b])
            pltpu.sync_copy(dstl_hbm.at[cid, pl.ds(c0, _K)], didxs[b])
            cps = [pltpu.async_copy(y_hbm.at[sidxs[b].at[j]],
                                    rowss[b].at[j], sem_g)
                   for j in range(_K)]
            for cp in cps:
                cp.wait()

        def fire_scatter(b):
            return [pltpu.async_copy(rowss[b].at[j], acc.at[didxs[b].at[j]],
                                     sems[b], add=True)
                    for j in range(_K)]

        gather_group(0, 0)

        def body(i, carry):
            g = 2 * i
            s0 = fire_scatter(0)
            gather_group(g + 1, 1)
            for cp in s0:
                cp.wait()
            s1 = fire_scatter(1)

            @pl.when(i + 1 < gpairs)
            def _():
                gather_group(g + 2, 0)

            for cp in s1:
                cp.wait()
            return carry

        lax.fori_loop(0, gpairs, body, 0)
        plsc.subcore_barrier()
        r0 = sid * opt
        pltpu.sync_copy(acc.at[pl.ds(r0, opt)], tbuf.at[pl.ds(0, opt)])
        pltpu.sync_copy(tbuf.at[pl.ds(0, opt)], out_hbm.at[cid, pl.ds(r0, opt)])

    return agg


def _elu(t):
    return jnp.where(t > 0, t, jnp.exp(jnp.minimum(t, 0.0)) - 1.0)


def _tc_dstloc(half, dst_ref, dstl_ref):
    d = dst_ref[...]
    dstl_ref[0] = jnp.where(d < half, d, half)
    dstl_ref[1] = jnp.where(d >= half, d - half, half)


def _tc_prep(dp_ref, x_ref, w1_ref, dinv_ref, y1_ref):
    deg = dp_ref[:, 0:1] + 1.0
    dinv = lax.rsqrt(deg)
    dinv_ref[...] = dinv
    xw = jnp.dot(x_ref[...], w1_ref[...], preferred_element_type=jnp.float32)
    y1_ref[...] = dinv * xw


def _tc_mid(a_ref, y_ref, dinv_ref, b_ref, g_ref, be_ref, wn_ref, yn_ref):
    dinv = dinv_ref[...]
    t = dinv * (a_ref[...] + y_ref[...]) + b_ref[...]
    m = jnp.mean(t, axis=-1, keepdims=True)
    d = t - m
    v = jnp.mean(d * d, axis=-1, keepdims=True)
    h = _elu(d * lax.rsqrt(v + 1e-5) * g_ref[...] + be_ref[...])
    yn_ref[...] = dinv * jnp.dot(h, wn_ref[...],
                                 preferred_element_type=jnp.float32)


def _tc_l4(a_ref, y_ref, dinv_ref, b4_ref, w5_ref, y5_ref):
    dinv = dinv_ref[...]
    t = dinv * (a_ref[...] + y_ref[...]) + b4_ref[...]
    h = _elu(t)
    y5 = dinv * jnp.sum(h * w5_ref[...], axis=-1, keepdims=True)
    y5_ref[...] = y5 + jnp.zeros((1, 8), jnp.float32)


def _tc_l5(a_ref, y_ref, dinv_ref, b5_ref, out_ref):
    t = dinv_ref[...] * (a_ref[:, 0:1] + y_ref[:, 0:1]) + b5_ref[...]
    out_ref[...] = _elu(t)


def _tc_call(body, out_shape, *args):
    return pl.pallas_call(body, out_shape=out_shape)(*args)


def kernel(x, edge_index, W1, b1, W2, b2, W3, b3, W4, b4, W5, b5,
           g2, be2, g3, be3):
    n, fin = x.shape
    e = edge_index.shape[1]
    f1 = W1.shape[1]
    f4 = W4.shape[1]
    n_pad = (n // 256 + 1) * 256
    half = n_pad // 2
    n_chunks = -(-e // _CHUNK)
    n_chunks = -(-n_chunks // (_NS * _K * 2)) * (_NS * _K * 2)
    e_pad = n_chunks * _CHUNK

    src = jnp.pad(edge_index[0], (0, e_pad - e), constant_values=n)
    dst = jnp.pad(edge_index[1], (0, e_pad - e), constant_values=n)
    src = src.reshape(n_chunks, _CHUNK)
    dst = dst.reshape(n_chunks, _CHUNK)

    x_pad = jnp.pad(x, ((0, n_pad - n), (0, 128 - fin)))
    w1_pad = jnp.pad(W1, ((0, 128 - fin), (0, 0)))

    zf = jnp.zeros((half // _NS + 1, f1), jnp.float32)
    z4 = jnp.zeros((half // _NS + 1, f4), jnp.float32)
    z8 = jnp.zeros((half // _NS + 1, 8), jnp.float32)

    agg_f = _make_agg(half, f1, n_chunks)
    agg_4 = _make_agg(half, f4, n_chunks)
    agg_8 = _make_agg(half, 8, n_chunks)

    dstl = _tc_call(functools.partial(_tc_dstloc, half),
                    jax.ShapeDtypeStruct((_NC, n_chunks, _CHUNK), jnp.int32),
                    dst)

    ones_tab = jnp.ones((n_pad, 8), jnp.float32)
    dp = agg_8(ones_tab, src, dstl, z8).reshape(n_pad, 8)
    dinv, y1 = _tc_call(
        _tc_prep,
        (jax.ShapeDtypeStruct((n_pad, 1), jnp.float32),
         jax.ShapeDtypeStruct((n_pad, f1), jnp.float32)),
        dp, x_pad, w1_pad)

    g2r, be2r = g2[None, :], be2[None, :]
    g3r, be3r = g3[None, :], be3[None, :]

    a1 = agg_f(y1, src, dstl, zf).reshape(n_pad, f1)
    y2 = _tc_call(_tc_mid, jax.ShapeDtypeStruct((n_pad, f1), jnp.float32),
                  a1, y1, dinv, b1[None, :], g2r, be2r, W2)
    a2 = agg_f(y2, src, dstl, zf).reshape(n_pad, f1)
    y3 = _tc_call(_tc_mid, jax.ShapeDtypeStruct((n_pad, f1), jnp.float32),
                  a2, y2, dinv, b2[None, :], g2r, be2r, W3)
    a3 = agg_f(y3, src, dstl, zf).reshape(n_pad, f1)
    y4 = _tc_call(_tc_mid, jax.ShapeDtypeStruct((n_pad, f4), jnp.float32),
                  a3, y3, dinv, b3[None, :], g3r, be3r, W4)
    a4 = agg_4(y4, src, dstl, z4).reshape(n_pad, f4)
    y5 = _tc_call(_tc_l4, jax.ShapeDtypeStruct((n_pad, 8), jnp.float32),
                  a4, y4, dinv, b4[None, :], W5.reshape(1, f4))
    a5 = agg_8(y5, src, dstl, z8).reshape(n_pad, 8)
    h5 = _tc_call(_tc_l5, jax.ShapeDtypeStruct((n_pad, 1), jnp.float32),
                  a5, y5, dinv, b5[None, :])

    r = 100 if n == 10000 else int(n ** 0.5)
    return h5[:n, 0].reshape(r, r)[None, :, :]


# speedup vs baseline: 9.6371x; 1.2361x over previous
import functools

import jax
import jax.numpy as jnp
from jax import lax
from jax.experimental import pallas as pl
from jax.experimental.pallas import tpu as pltpu
from jax.experimental.pallas import tpu_sc as plsc

_SC_PARAMS = pltpu.CompilerParams(use_tc_tiling_on_sc=False)

_NC, _NS = 2, 16
_CHUNK = 128
_K = 4


def _make_mesh():
    return plsc.VectorSubcoreMesh(
        core_axis_name="c", subcore_axis_name="s",
        num_cores=_NC, num_subcores=_NS)


def _make_agg(half, f, n_chunks):
    n_acc = half + 16
    cpt = n_chunks // _NS
    groups = cpt // _K
    gpairs = groups // 2
    rpt = n_acc // _NS
    opt = half // _NS

    @functools.partial(
        pl.kernel,
        out_type=jax.ShapeDtypeStruct((_NC, half, f), jnp.float32),
        mesh=_make_mesh(),
        compiler_params=_SC_PARAMS,
        scratch_types=[
            pltpu.VMEM((_K, _CHUNK), jnp.int32),
            pltpu.VMEM((_K, _CHUNK), jnp.int32),
            pltpu.VMEM((_K, _CHUNK), jnp.int32),
            pltpu.VMEM((_K, _CHUNK), jnp.int32),
            pltpu.VMEM((_K, _CHUNK, f), jnp.float32),
            pltpu.VMEM((_K, _CHUNK, f), jnp.float32),
            pltpu.VMEM((rpt, f), jnp.float32),
            pltpu.VMEM_SHARED((n_acc, f), jnp.float32),
            pltpu.SemaphoreType.DMA,
            pltpu.SemaphoreType.DMA,
            pltpu.SemaphoreType.DMA,
        ],
    )
    def agg(y_hbm, src_hbm, dstl_hbm, zero_hbm, out_hbm,
            sidx0, sidx1, didx0, didx1, rows0, rows1, tbuf, acc,
            sem_g, sem_s0, sem_s1):
        cid = lax.axis_index("c")
        sid = lax.axis_index("s")
        pltpu.sync_copy(zero_hbm, tbuf)
        pltpu.sync_copy(tbuf, acc.at[pl.ds(sid * rpt, rpt)])
        plsc.subcore_barrier()
        sidxs, didxs, rowss = (sidx0, sidx1), (didx0, didx1), (rows0, rows1)
        sems = (sem_s0, sem_s1)

        def gather_group(g, b):
            c0 = sid * cpt + g * _K
            pltpu.sync_copy(src_hbm.at[pl.ds(c0, _K)], sidxs[b])
            pltpu.sync_copy(dstl_hbm.at[cid, pl.ds(c0, _K)], didxs[b])
            cps = [pltpu.async_copy(y_hbm.at[sidxs[b].at[j]],
                                    rowss[b].at[j], sem_g)
                   for j in range(_K)]
            for cp in cps:
                cp.wait()

        def fire_scatter(b):
            return [pltpu.async_copy(rowss[b].at[j], acc.at[didxs[b].at[j]],
                                     sems[b], add=True)
                    for j in range(_K)]

        gather_group(0, 0)

        def body(i, carry):
            g = 2 * i
            s0 = fire_scatter(0)
            gather_group(g + 1, 1)
            for cp in s0:
                cp.wait()
            s1 = fire_scatter(1)

            @pl.when(i + 1 < gpairs)
            def _():
                gather_group(g + 2, 0)

            for cp in s1:
                cp.wait()
            return carry

        lax.fori_loop(0, gpairs, body, 0)
        plsc.subcore_barrier()
        r0 = sid * opt
        pltpu.sync_copy(acc.at[pl.ds(r0, opt)], tbuf.at[pl.ds(0, opt)])
        pltpu.sync_copy(tbuf.at[pl.ds(0, opt)], out_hbm.at[cid, pl.ds(r0, opt)])

    return agg


def _elu(t):
    return jnp.where(t > 0, t, jnp.exp(jnp.minimum(t, 0.0)) - 1.0)


def _tc_dstloc(half, dst_ref, dstl_ref):
    d = dst_ref[...]
    dummy = half + (d & 15)
    dstl_ref[0] = jnp.where(d < half, d, dummy)
    dstl_ref[1] = jnp.where(d >= half, d - half, dummy)


def _tc_prep(dp_ref, x_ref, w1_ref, dinv_ref, y1_ref):
    deg = dp_ref[:, 0:1] + 1.0
    dinv = lax.rsqrt(deg)
    dinv_ref[...] = dinv
    xw = jnp.dot(x_ref[...], w1_ref[...], preferred_element_type=jnp.float32)
    y1_ref[...] = dinv * xw


def _tc_mid(a_ref, y_ref, dinv_ref, b_ref, g_ref, be_ref, wn_ref, yn_ref):
    dinv = dinv_ref[...]
    t = dinv * (a_ref[...] + y_ref[...]) + b_ref[...]
    m = jnp.mean(t, axis=-1, keepdims=True)
    d = t - m
    v = jnp.mean(d * d, axis=-1, keepdims=True)
    h = _elu(d * lax.rsqrt(v + 1e-5) * g_ref[...] + be_ref[...])
    yn_ref[...] = dinv * jnp.dot(h, wn_ref[...],
                                 preferred_element_type=jnp.float32)


def _tc_l4(a_ref, y_ref, dinv_ref, b4_ref, w5_ref, y5_ref):
    dinv = dinv_ref[...]
    t = dinv * (a_ref[...] + y_ref[...]) + b4_ref[...]
    h = _elu(t)
    y5 = dinv * jnp.sum(h * w5_ref[...], axis=-1, keepdims=True)
    y5_ref[...] = y5 + jnp.zeros((1, 8), jnp.float32)


def _tc_l5(a_ref, y_ref, dinv_ref, b5_ref, out_ref):
    t = dinv_ref[...] * (a_ref[:, 0:1] + y_ref[:, 0:1]) + b5_ref[...]
    out_ref[...] = _elu(t)


def _tc_call(body, out_shape, *args):
    return pl.pallas_call(body, out_shape=out_shape)(*args)


def kernel(x, edge_index, W1, b1, W2, b2, W3, b3, W4, b4, W5, b5,
           g2, be2, g3, be3):
    n, fin = x.shape
    e = edge_index.shape[1]
    f1 = W1.shape[1]
    f4 = W4.shape[1]
    n_pad = (n // 256 + 1) * 256
    half = n_pad // 2
    n_chunks = -(-e // _CHUNK)
    n_chunks = -(-n_chunks // (_NS * _K * 2)) * (_NS * _K * 2)
    e_pad = n_chunks * _CHUNK

    src = jnp.pad(edge_index[0], (0, e_pad - e), constant_values=n)
    dst = jnp.pad(edge_index[1], (0, e_pad - e), constant_values=n)
    src = src.reshape(n_chunks, _CHUNK)
    dst = dst.reshape(n_chunks, _CHUNK)

    x_pad = jnp.pad(x, ((0, n_pad - n), (0, 128 - fin)))
    w1_pad = jnp.pad(W1, ((0, 128 - fin), (0, 0)))

    zf = jnp.zeros((half // _NS + 1, f1), jnp.float32)
    z4 = jnp.zeros((half // _NS + 1, f4), jnp.float32)
    z8 = jnp.zeros((half // _NS + 1, 8), jnp.float32)

    agg_f = _make_agg(half, f1, n_chunks)
    agg_4 = _make_agg(half, f4, n_chunks)
    agg_8 = _make_agg(half, 8, n_chunks)

    dstl = _tc_call(functools.partial(_tc_dstloc, half),
                    jax.ShapeDtypeStruct((_NC, n_chunks, _CHUNK), jnp.int32),
                    dst)

    ones_tab = jnp.ones((n_pad, 8), jnp.float32)
    dp = agg_8(ones_tab, src, dstl, z8).reshape(n_pad, 8)
    dinv, y1 = _tc_call(
        _tc_prep,
        (jax.ShapeDtypeStruct((n_pad, 1), jnp.float32),
         jax.ShapeDtypeStruct((n_pad, f1), jnp.float32)),
        dp, x_pad, w1_pad)

    g2r, be2r = g2[None, :], be2[None, :]
    g3r, be3r = g3[None, :], be3[None, :]

    a1 = agg_f(y1, src, dstl, zf).reshape(n_pad, f1)
    y2 = _tc_call(_tc_mid, jax.ShapeDtypeStruct((n_pad, f1), jnp.float32),
                  a1, y1, dinv, b1[None, :], g2r, be2r, W2)
    a2 = agg_f(y2, src, dstl, zf).reshape(n_pad, f1)
    y3 = _tc_call(_tc_mid, jax.ShapeDtypeStruct((n_pad, f1), jnp.float32),
                  a2, y2, dinv, b2[None, :], g2r, be2r, W3)
    a3 = agg_f(y3, src, dstl, zf).reshape(n_pad, f1)
    y4 = _tc_call(_tc_mid, jax.ShapeDtypeStruct((n_pad, f4), jnp.float32),
                  a3, y3, dinv, b3[None, :], g3r, be3r, W4)
    a4 = agg_4(y4, src, dstl, z4).reshape(n_pad, f4)
    y5 = _tc_call(_tc_l4, jax.ShapeDtypeStruct((n_pad, 8), jnp.float32),
                  a4, y4, dinv, b4[None, :], W5.reshape(1, f4))
    a5 = agg_8(y5, src, dstl, z8).reshape(n_pad, 8)
    h5 = _tc_call(_tc_l5, jax.ShapeDtypeStruct((n_pad, 1), jnp.float32),
                  a5, y5, dinv, b5[None, :])

    r = 100 if n == 10000 else int(n ** 0.5)
    return h5[:n, 0].reshape(r, r)[None, :, :]


# speedup vs baseline: 9.6438x; 1.0007x over previous
import functools

import jax
import jax.numpy as jnp
from jax import lax
from jax.experimental import pallas as pl
from jax.experimental.pallas import tpu as pltpu
from jax.experimental.pallas import tpu_sc as plsc

_SC_PARAMS = pltpu.CompilerParams(use_tc_tiling_on_sc=False)

_NC, _NS = 2, 16
_CHUNK = 128
_K = 4


def _make_mesh():
    return plsc.VectorSubcoreMesh(
        core_axis_name="c", subcore_axis_name="s",
        num_cores=_NC, num_subcores=_NS)


def _make_agg(half, f, n_chunks):
    n_acc = half + 128
    cpt = n_chunks // _NS
    groups = cpt // _K
    gpairs = groups // 2
    rpt = n_acc // _NS
    opt = half // _NS

    @functools.partial(
        pl.kernel,
        out_type=jax.ShapeDtypeStruct((_NC, half, f), jnp.float32),
        mesh=_make_mesh(),
        compiler_params=_SC_PARAMS,
        scratch_types=[
            pltpu.VMEM((_K, _CHUNK), jnp.int32),
            pltpu.VMEM((_K, _CHUNK), jnp.int32),
            pltpu.VMEM((_K, _CHUNK), jnp.int32),
            pltpu.VMEM((_K, _CHUNK), jnp.int32),
            pltpu.VMEM((_K, _CHUNK, f), jnp.float32),
            pltpu.VMEM((_K, _CHUNK, f), jnp.float32),
            pltpu.VMEM((rpt, f), jnp.float32),
            pltpu.VMEM_SHARED((n_acc, f), jnp.float32),
            pltpu.SemaphoreType.DMA,
            pltpu.SemaphoreType.DMA,
            pltpu.SemaphoreType.DMA,
        ],
    )
    def agg(y_hbm, src_hbm, dstl_hbm, zero_hbm, out_hbm,
            sidx0, sidx1, didx0, didx1, rows0, rows1, tbuf, acc,
            sem_g, sem_s0, sem_s1):
        cid = lax.axis_index("c")
        sid = lax.axis_index("s")
        pltpu.sync_copy(zero_hbm, tbuf)
        pltpu.sync_copy(tbuf, acc.at[pl.ds(sid * rpt, rpt)])
        plsc.subcore_barrier()
        sidxs, didxs, rowss = (sidx0, sidx1), (didx0, didx1), (rows0, rows1)
        sems = (sem_s0, sem_s1)

        def gather_group(g, b):
            c0 = sid * cpt + g * _K
            pltpu.sync_copy(src_hbm.at[pl.ds(c0, _K)], sidxs[b])
            pltpu.sync_copy(dstl_hbm.at[cid, pl.ds(c0, _K)], didxs[b])
            cps = [pltpu.async_copy(y_hbm.at[sidxs[b].at[j]],
                                    rowss[b].at[j], sem_g)
                   for j in range(_K)]
            for cp in cps:
                cp.wait()

        def fire_scatter(b):
            return [pltpu.async_copy(rowss[b].at[j], acc.at[didxs[b].at[j]],
                                     sems[b], add=True)
                    for j in range(_K)]

        gather_group(0, 0)

        def body(i, carry):
            g = 2 * i
            s0 = fire_scatter(0)
            gather_group(g + 1, 1)
            for cp in s0:
                cp.wait()
            s1 = fire_scatter(1)

            @pl.when(i + 1 < gpairs)
            def _():
                gather_group(g + 2, 0)

            for cp in s1:
                cp.wait()
            return carry

        lax.fori_loop(0, gpairs, body, 0)
        plsc.subcore_barrier()
        r0 = sid * opt
        pltpu.sync_copy(acc.at[pl.ds(r0, opt)], tbuf.at[pl.ds(0, opt)])
        pltpu.sync_copy(tbuf.at[pl.ds(0, opt)], out_hbm.at[cid, pl.ds(r0, opt)])

    return agg


def _elu(t):
    return jnp.where(t > 0, t, jnp.exp(jnp.minimum(t, 0.0)) - 1.0)


def _tc_dstloc(half, dst_ref, dstl_ref):
    d = dst_ref[...]
    dummy = half + (d & 127)
    dstl_ref[0] = jnp.where(d < half, d, dummy)
    dstl_ref[1] = jnp.where(d >= half, d - half, dummy)


def _tc_prep(dp_ref, x_ref, w1_ref, dinv_ref, y1_ref):
    deg = dp_ref[:, 0:1] + 1.0
    dinv = lax.rsqrt(deg)
    dinv_ref[...] = dinv
    xw = jnp.dot(x_ref[...], w1_ref[...], preferred_element_type=jnp.float32)
    y1_ref[...] = dinv * xw


def _tc_mid(a_ref, y_ref, dinv_ref, b_ref, g_ref, be_ref, wn_ref, yn_ref):
    dinv = dinv_ref[...]
    t = dinv * (a_ref[...] + y_ref[...]) + b_ref[...]
    m = jnp.mean(t, axis=-1, keepdims=True)
    d = t - m
    v = jnp.mean(d * d, axis=-1, keepdims=True)
    h = _elu(d * lax.rsqrt(v + 1e-5) * g_ref[...] + be_ref[...])
    yn_ref[...] = dinv * jnp.dot(h, wn_ref[...],
                                 preferred_element_type=jnp.float32)


def _tc_l4(a_ref, y_ref, dinv_ref, b4_ref, w5_ref, y5_ref):
    dinv = dinv_ref[...]
    t = dinv * (a_ref[...] + y_ref[...]) + b4_ref[...]
    h = _elu(t)
    y5 = dinv * jnp.sum(h * w5_ref[...], axis=-1, keepdims=True)
    y5_ref[...] = y5 + jnp.zeros((1, 8), jnp.float32)


def _tc_l5(a_ref, y_ref, dinv_ref, b5_ref, out_ref):
    t = dinv_ref[...] * (a_ref[:, 0:1] + y_ref[:, 0:1]) + b5_ref[...]
    out_ref[...] = _elu(t)


def _tc_call(body, out_shape, *args):
    return pl.pallas_call(body, out_shape=out_shape)(*args)


def kernel(x, edge_index, W1, b1, W2, b2, W3, b3, W4, b4, W5, b5,
           g2, be2, g3, be3):
    n, fin = x.shape
    e = edge_index.shape[1]
    f1 = W1.shape[1]
    f4 = W4.shape[1]
    n_pad = (n // 256 + 1) * 256
    half = n_pad // 2
    n_chunks = -(-e // _CHUNK)
    n_chunks = -(-n_chunks // (_NS * _K * 2)) * (_NS * _K * 2)
    e_pad = n_chunks * _CHUNK

    src = jnp.pad(edge_index[0], (0, e_pad - e), constant_values=n)
    dst = jnp.pad(edge_index[1], (0, e_pad - e), constant_values=n)
    src = src.reshape(n_chunks, _CHUNK)
    dst = dst.reshape(n_chunks, _CHUNK)

    x_pad = jnp.pad(x, ((0, n_pad - n), (0, 128 - fin)))
    w1_pad = jnp.pad(W1, ((0, 128 - fin), (0, 0)))

    zf = jnp.zeros((half // _NS + 8, f1), jnp.float32)
    z4 = jnp.zeros((half // _NS + 8, f4), jnp.float32)
    z8 = jnp.zeros((half // _NS + 8, 8), jnp.float32)

    agg_f = _make_agg(half, f1, n_chunks)
    agg_4 = _make_agg(half, f4, n_chunks)
    agg_8 = _make_agg(half, 8, n_chunks)

    dstl = _tc_call(functools.partial(_tc_dstloc, half),
                    jax.ShapeDtypeStruct((_NC, n_chunks, _CHUNK), jnp.int32),
                    dst)

    ones_tab = jnp.ones((n_pad, 8), jnp.float32)
    dp = agg_8(ones_tab, src, dstl, z8).reshape(n_pad, 8)
    dinv, y1 = _tc_call(
        _tc_prep,
        (jax.ShapeDtypeStruct((n_pad, 1), jnp.float32),
         jax.ShapeDtypeStruct((n_pad, f1), jnp.float32)),
        dp, x_pad, w1_pad)

    g2r, be2r = g2[None, :], be2[None, :]
    g3r, be3r = g3[None, :], be3[None, :]

    a1 = agg_f(y1, src, dstl, zf).reshape(n_pad, f1)
    y2 = _tc_call(_tc_mid, jax.ShapeDtypeStruct((n_pad, f1), jnp.float32),
                  a1, y1, dinv, b1[None, :], g2r, be2r, W2)
    a2 = agg_f(y2, src, dstl, zf).reshape(n_pad, f1)
    y3 = _tc_call(_tc_mid, jax.ShapeDtypeStruct((n_pad, f1), jnp.float32),
                  a2, y2, dinv, b2[None, :], g2r, be2r, W3)
    a3 = agg_f(y3, src, dstl, zf).reshape(n_pad, f1)
    y4 = _tc_call(_tc_mid, jax.ShapeDtypeStruct((n_pad, f4), jnp.float32),
                  a3, y3, dinv, b3[None, :], g3r, be3r, W4)
    a4 = agg_4(y4, src, dstl, z4).reshape(n_pad, f4)
    y5 = _tc_call(_tc_l4, jax.ShapeDtypeStruct((n_pad, 8), jnp.float32),
                  a4, y4, dinv, b4[None, :], W5.reshape(1, f4))
    a5 = agg_8(y5, src, dstl, z8).reshape(n_pad, 8)
    h5 = _tc_call(_tc_l5, jax.ShapeDtypeStruct((n_pad, 1), jnp.float32),
                  a5, y5, dinv, b5[None, :])

    r = 100 if n == 10000 else int(n ** 0.5)
    return h5[:n, 0].reshape(r, r)[None, :, :]


# speedup vs baseline: 9.8559x; 1.0220x over previous
import functools

import jax
import jax.numpy as jnp
from jax import lax
from jax.experimental import pallas as pl
from jax.experimental.pallas import tpu as pltpu
from jax.experimental.pallas import tpu_sc as plsc

_SC_PARAMS = pltpu.CompilerParams(use_tc_tiling_on_sc=False)

_NC, _NS = 2, 16
_CHUNK = 128
_K = 5


def _make_mesh():
    return plsc.VectorSubcoreMesh(
        core_axis_name="c", subcore_axis_name="s",
        num_cores=_NC, num_subcores=_NS)


def _make_agg(half, f, n_chunks):
    n_acc = half + 128
    cpt = n_chunks // _NS
    groups = cpt // _K
    gpairs = groups // 2
    rpt = n_acc // _NS
    opt = half // _NS

    @functools.partial(
        pl.kernel,
        out_type=jax.ShapeDtypeStruct((_NC, half, f), jnp.float32),
        mesh=_make_mesh(),
        compiler_params=_SC_PARAMS,
        scratch_types=[
            pltpu.VMEM((_K, _CHUNK), jnp.int32),
            pltpu.VMEM((_K, _CHUNK), jnp.int32),
            pltpu.VMEM((_K, _CHUNK), jnp.int32),
            pltpu.VMEM((_K, _CHUNK), jnp.int32),
            pltpu.VMEM((_K, _CHUNK, f), jnp.float32),
            pltpu.VMEM((_K, _CHUNK, f), jnp.float32),
            pltpu.VMEM((rpt, f), jnp.float32),
            pltpu.VMEM_SHARED((n_acc, f), jnp.float32),
            pltpu.SemaphoreType.DMA,
            pltpu.SemaphoreType.DMA,
            pltpu.SemaphoreType.DMA,
        ],
    )
    def agg(y_hbm, src_hbm, dstl_hbm, zero_hbm, out_hbm,
            sidx0, sidx1, didx0, didx1, rows0, rows1, tbuf, acc,
            sem_g, sem_s0, sem_s1):
        cid = lax.axis_index("c")
        sid = lax.axis_index("s")
        pltpu.sync_copy(zero_hbm, tbuf)
        pltpu.sync_copy(tbuf, acc.at[pl.ds(sid * rpt, rpt)])
        plsc.subcore_barrier()
        sidxs, didxs, rowss = (sidx0, sidx1), (didx0, didx1), (rows0, rows1)
        sems = (sem_s0, sem_s1)

        def gather_group(g, b):
            c0 = sid * cpt + g * _K
            pltpu.sync_copy(src_hbm.at[pl.ds(c0, _K)], sidxs[b])
            pltpu.sync_copy(dstl_hbm.at[cid, pl.ds(c0, _K)], didxs[b])
            cps = [pltpu.async_copy(y_hbm.at[sidxs[b].at[j]],
                                    rowss[b].at[j], sem_g)
                   for j in range(_K)]
            for cp in cps:
                cp.wait()

        def fire_scatter(b):
            return [pltpu.async_copy(rowss[b].at[j], acc.at[didxs[b].at[j]],
                                     sems[b], add=True)
                    for j in range(_K)]

        gather_group(0, 0)

        def body(i, carry):
            g = 2 * i
            s0 = fire_scatter(0)
            gather_group(g + 1, 1)
            for cp in s0:
                cp.wait()
            s1 = fire_scatter(1)

            @pl.when(i + 1 < gpairs)
            def _():
                gather_group(g + 2, 0)

            for cp in s1:
                cp.wait()
            return carry

        lax.fori_loop(0, gpairs, body, 0)
        plsc.subcore_barrier()
        r0 = sid * opt
        pltpu.sync_copy(acc.at[pl.ds(r0, opt)], tbuf.at[pl.ds(0, opt)])
        pltpu.sync_copy(tbuf.at[pl.ds(0, opt)], out_hbm.at[cid, pl.ds(r0, opt)])

    return agg


def _elu(t):
    return jnp.where(t > 0, t, jnp.exp(jnp.minimum(t, 0.0)) - 1.0)


def _tc_dstloc(half, dst_ref, dstl_ref):
    d = dst_ref[...]
    dummy = half + (d & 127)
    dstl_ref[0] = jnp.where(d < half, d, dummy)
    dstl_ref[1] = jnp.where(d >= half, d - half, dummy)


def _tc_prep(dp_ref, x_ref, w1_ref, dinv_ref, y1_ref):
    deg = dp_ref[:, 0:1] + 1.0
    dinv = lax.rsqrt(deg)
    dinv_ref[...] = dinv
    xw = jnp.dot(x_ref[...], w1_ref[...], preferred_element_type=jnp.float32)
    y1_ref[...] = dinv * xw


def _tc_mid(a_ref, y_ref, dinv_ref, b_ref, g_ref, be_ref, wn_ref, yn_ref):
    dinv = dinv_ref[...]
    t = dinv * (a_ref[...] + y_ref[...]) + b_ref[...]
    m = jnp.mean(t, axis=-1, keepdims=True)
    d = t - m
    v = jnp.mean(d * d, axis=-1, keepdims=True)
    h = _elu(d * lax.rsqrt(v + 1e-5) * g_ref[...] + be_ref[...])
    yn_ref[...] = dinv * jnp.dot(h, wn_ref[...],
                                 preferred_element_type=jnp.float32)


def _tc_l4(a_ref, y_ref, dinv_ref, b4_ref, w5_ref, y5_ref):
    dinv = dinv_ref[...]
    t = dinv * (a_ref[...] + y_ref[...]) + b4_ref[...]
    h = _elu(t)
    y5 = dinv * jnp.sum(h * w5_ref[...], axis=-1, keepdims=True)
    y5_ref[...] = y5 + jnp.zeros((1, 8), jnp.float32)


def _tc_l5(a_ref, y_ref, dinv_ref, b5_ref, out_ref):
    t = dinv_ref[...] * (a_ref[:, 0:1] + y_ref[:, 0:1]) + b5_ref[...]
    out_ref[...] = _elu(t)


def _tc_call(body, out_shape, *args):
    return pl.pallas_call(body, out_shape=out_shape)(*args)


def kernel(x, edge_index, W1, b1, W2, b2, W3, b3, W4, b4, W5, b5,
           g2, be2, g3, be3):
    n, fin = x.shape
    e = edge_index.shape[1]
    f1 = W1.shape[1]
    f4 = W4.shape[1]
    n_pad = (n // 256 + 1) * 256
    half = n_pad // 2
    n_chunks = -(-e // _CHUNK)
    n_chunks = -(-n_chunks // (_NS * _K * 2)) * (_NS * _K * 2)
    e_pad = n_chunks * _CHUNK

    src = jnp.pad(edge_index[0], (0, e_pad - e), constant_values=n)
    dst = jnp.pad(edge_index[1], (0, e_pad - e), constant_values=n)
    src = src.reshape(n_chunks, _CHUNK)
    dst = dst.reshape(n_chunks, _CHUNK)

    x_pad = jnp.pad(x, ((0, n_pad - n), (0, 128 - fin)))
    w1_pad = jnp.pad(W1, ((0, 128 - fin), (0, 0)))

    zf = jnp.zeros((half // _NS + 8, f1), jnp.float32)
    z4 = jnp.zeros((half // _NS + 8, f4), jnp.float32)
    z8 = jnp.zeros((half // _NS + 8, 8), jnp.float32)

    agg_f = _make_agg(half, f1, n_chunks)
    agg_4 = _make_agg(half, f4, n_chunks)
    agg_8 = _make_agg(half, 8, n_chunks)

    dstl = _tc_call(functools.partial(_tc_dstloc, half),
                    jax.ShapeDtypeStruct((_NC, n_chunks, _CHUNK), jnp.int32),
                    dst)

    ones_tab = jnp.ones((n_pad, 8), jnp.float32)
    dp = agg_8(ones_tab, src, dstl, z8).reshape(n_pad, 8)
    dinv, y1 = _tc_call(
        _tc_prep,
        (jax.ShapeDtypeStruct((n_pad, 1), jnp.float32),
         jax.ShapeDtypeStruct((n_pad, f1), jnp.float32)),
        dp, x_pad, w1_pad)

    g2r, be2r = g2[None, :], be2[None, :]
    g3r, be3r = g3[None, :], be3[None, :]

    a1 = agg_f(y1, src, dstl, zf).reshape(n_pad, f1)
    y2 = _tc_call(_tc_mid, jax.ShapeDtypeStruct((n_pad, f1), jnp.float32),
                  a1, y1, dinv, b1[None, :], g2r, be2r, W2)
    a2 = agg_f(y2, src, dstl, zf).reshape(n_pad, f1)
    y3 = _tc_call(_tc_mid, jax.ShapeDtypeStruct((n_pad, f1), jnp.float32),
                  a2, y2, dinv, b2[None, :], g2r, be2r, W3)
    a3 = agg_f(y3, src, dstl, zf).reshape(n_pad, f1)
    y4 = _tc_call(_tc_mid, jax.ShapeDtypeStruct((n_pad, f4), jnp.float32),
                  a3, y3, dinv, b3[None, :], g3r, be3r, W4)
    a4 = agg_4(y4, src, dstl, z4).reshape(n_pad, f4)
    y5 = _tc_call(_tc_l4, jax.ShapeDtypeStruct((n_pad, 8), jnp.float32),
                  a4, y4, dinv, b4[None, :], W5.reshape(1, f4))
    a5 = agg_8(y5, src, dstl, z8).reshape(n_pad, 8)
    h5 = _tc_call(_tc_l5, jax.ShapeDtypeStruct((n_pad, 1), jnp.float32),
                  a5, y5, dinv, b5[None, :])

    r = 100 if n == 10000 else int(n ** 0.5)
    return h5[:n, 0].reshape(r, r)[None, :, :]
